# baseline (speedup 1.0000x reference)
import functools

import jax
import jax.numpy as jnp
from jax import lax
from jax.experimental import pallas as pl
from jax.experimental.pallas import tpu as pltpu
from jax.experimental.pallas import tpu_sc as plsc

MIN_NORM = 1e-15
EPS = 4e-3
MAX_NORM = 1e6
N_NODES = 10000
N_EDGES = 160000
D = 256

ROWS = 1000


def _col_mask(v):
    col = lax.broadcasted_iota(jnp.int32, v.shape, dimension=v.ndim - 1)
    return col == 0


def _zero_col0(v):
    return jnp.where(_col_mask(v), 0.0, v)


def _sq_rest(v):
    vz = _zero_col0(v)
    return jnp.sum(vz * vz, axis=-1, keepdims=True)


def _dot_rest(a, b):
    p = _zero_col0(a) * _zero_col0(b)
    return jnp.sum(p, axis=-1, keepdims=True)


def _col0(v):
    return v[..., 0:1]


def _set_col0(v, s):
    return jnp.where(_col_mask(v), s, v)


def _cosh_sinh(t):
    e = jnp.exp(t)
    ei = 1.0 / e
    return 0.5 * (e + ei), 0.5 * (e - ei)


def _arccosh(t):
    return jnp.log(t + jnp.sqrt(t * t - 1.0))


def _proj(v):
    x0 = jnp.sqrt(jnp.clip(1.0 + _sq_rest(v), EPS, None))
    return _set_col0(v, x0)


def _expmap0(u):
    n = jnp.clip(jnp.sqrt(_sq_rest(u)), MIN_NORM, None)
    ch, sh = _cosh_sinh(n)
    y = (sh / n) * _zero_col0(u)
    return _proj(_set_col0(y, ch))


def _logmap0(x):
    yn = jnp.clip(jnp.sqrt(_sq_rest(x)), MIN_NORM, None)
    theta = jnp.clip(_col0(x), 1.0 + EPS, None)
    r = (_arccosh(theta) / yn) * _zero_col0(x)
    return _set_col0(r, 0.0)


def _hyp_bias(b_row):
    return _proj(_expmap0(_zero_col0(b_row)))


def _mobius_add(x, yb):
    u = _logmap0(yb)
    x0 = _col0(x)
    y_norm = jnp.clip(jnp.sqrt(_sq_rest(x)), MIN_NORM, None)
    y_unit = _zero_col0(x) / y_norm
    v = _set_col0((1.0 - x0) * y_unit, -y_norm)
    alpha = jnp.sum(y_unit * _zero_col0(u), axis=-1, keepdims=True)
    res = u - alpha * v
    ux = _dot_rest(x, res)
    u0 = ux / jnp.clip(x0, EPS, None)
    res = _set_col0(res, u0)
    mdot = _sq_rest(res) - _col0(res) * _col0(res)
    normu = jnp.sqrt(jnp.clip(mdot, EPS, None))
    normu = jnp.clip(normu, None, MAX_NORM)
    theta = jnp.clip(normu, MIN_NORM, None)
    ch, sh = _cosh_sinh(theta)
    return _proj(ch * x + (sh / theta) * res)


def _hyp_linear_to_tan(x_hyp, Wt, b_row):
    u = _logmap0(x_hyp)
    mu = jnp.dot(u, Wt, preferred_element_type=jnp.float32)
    res = _proj(_expmap0(mu))
    res = _proj(_mobius_add(res, _hyp_bias(b_row)))
    return _logmap0(res)


def _agg_to_hyp(agg):
    h = _proj(_expmap0(agg))
    xt = jax.nn.relu(_logmap0(h))
    xt = _set_col0(xt, 0.0)
    return _proj(_expmap0(xt))


def _tc_pre_body(x_ref, wt_ref, b_ref, o0_ref, o1_ref):
    x = x_ref[...]
    x_hyp = _expmap0(_zero_col0(x))
    xt = _hyp_linear_to_tan(x_hyp, wt_ref[...], b_ref[...])
    o0_ref[...] = xt[:, :128]
    o1_ref[...] = xt[:, 128:]


def _tc_mid_body(a0_ref, a1_ref, deg_ref, wt_ref, b_ref, o0_ref, o1_ref):
    agg = jnp.concatenate([a0_ref[...], a1_ref[...]], axis=-1)
    deg = deg_ref[0, :, 0:1] + deg_ref[1, :, 0:1]
    agg = agg / jnp.clip(deg, 1.0, None)
    h = _agg_to_hyp(agg)
    xt = _hyp_linear_to_tan(h, wt_ref[...], b_ref[...])
    o0_ref[...] = xt[:, :128]
    o1_ref[...] = xt[:, 128:]


def _tc_post_body(a0_ref, a1_ref, deg_ref, o_ref):
    agg = jnp.concatenate([a0_ref[...], a1_ref[...]], axis=-1)
    deg = deg_ref[0, :, 0:1] + deg_ref[1, :, 0:1]
    agg = agg / jnp.clip(deg, 1.0, None)
    h = _agg_to_hyp(agg)
    out = _logmap0(h)
    o_ref[...] = _set_col0(out, 0.0)


def _row_spec(width):
    return pl.BlockSpec((ROWS, width), lambda i: (i, 0))


def _full_spec(shape):
    return pl.BlockSpec(shape, lambda i: tuple(0 for _ in shape))


def _deg_spec():
    return pl.BlockSpec((2, ROWS, 128), lambda i: (0, i, 0))


def _tc_pre(x, Wt, b_row):
    grid = N_NODES // ROWS
    return pl.pallas_call(
        _tc_pre_body,
        grid=(grid,),
        in_specs=[_row_spec(D), _full_spec((D, D)), _full_spec((1, D))],
        out_specs=[_row_spec(128), _row_spec(128)],
        out_shape=[jax.ShapeDtypeStruct((N_NODES, 128), jnp.float32)] * 2,
    )(x, Wt, b_row)


def _tc_mid(a0, a1, deg, Wt, b_row):
    grid = N_NODES // ROWS
    return pl.pallas_call(
        _tc_mid_body,
        grid=(grid,),
        in_specs=[_row_spec(128), _row_spec(128), _deg_spec(),
                  _full_spec((D, D)), _full_spec((1, D))],
        out_specs=[_row_spec(128), _row_spec(128)],
        out_shape=[jax.ShapeDtypeStruct((N_NODES, 128), jnp.float32)] * 2,
    )(a0, a1, deg, Wt, b_row)


def _tc_post(a0, a1, deg):
    grid = N_NODES // ROWS
    return pl.pallas_call(
        _tc_post_body,
        grid=(grid,),
        in_specs=[_row_spec(128), _row_spec(128), _deg_spec()],
        out_specs=_row_spec(D),
        out_shape=jax.ShapeDtypeStruct((N_NODES, D), jnp.float32),
    )(a0, a1, deg)


CH = 128
GC = 16
NG = 5
EPT = NG * GC * CH
N_EPAD = 16 * EPT
ACC_ROWS = N_NODES + 8
R_MAIN = 624
R_TAIL = N_NODES - 16 * R_MAIN


def _sc_agg_kernel(with_deg):
    mesh = plsc.VectorSubcoreMesh(core_axis_name="c", subcore_axis_name="s",
                                  num_cores=2, num_subcores=16)
    out_type = [jax.ShapeDtypeStruct((N_NODES, 128), jnp.float32)] * 2
    scratch = [
        pltpu.VMEM_SHARED((ACC_ROWS, 128), jnp.float32),
        pltpu.VMEM((GC, CH), jnp.int32),
        pltpu.VMEM((GC, CH), jnp.int32),
        pltpu.VMEM((CH, 128), jnp.float32),
        pltpu.VMEM((CH, 128), jnp.float32),
        pltpu.SemaphoreType.DMA,
        pltpu.SemaphoreType.DMA,
        pltpu.SemaphoreType.DMA,
        pltpu.SemaphoreType.DMA,
    ]
    if with_deg:
        out_type.append(jax.ShapeDtypeStruct((2, N_NODES, 128), jnp.float32))

    @functools.partial(pl.kernel, mesh=mesh, out_type=out_type,
                       scratch_types=scratch)
    def body(t0, t1, src3d, dst3d, zf, ones, out0, out1, *rest):
        if with_deg:
            dout, acc, sv, dv, rv0, rv1, sg0, sg1, ss0, ss1 = rest
        else:
            acc, sv, dv, rv0, rv1, sg0, sg1, ss0, ss1 = rest
            dout = None
        cid = lax.axis_index("c")
        sid = lax.axis_index("s")

        def rows_copy(a, b):
            r0 = sid * R_MAIN
            pltpu.sync_copy(a.at[pl.ds(r0, R_MAIN)], b.at[pl.ds(r0, R_MAIN)])

            @pl.when(sid == 15)
            def _():
                t0_ = 16 * R_MAIN
                pltpu.sync_copy(a.at[pl.ds(t0_, R_TAIL)],
                                b.at[pl.ds(t0_, R_TAIL)])

        rows_copy(zf, acc)

        @pl.when(sid == 15)
        def _():
            pltpu.sync_copy(zf.at[pl.ds(0, 8)],
                            acc.at[pl.ds(N_NODES, 8)])

        if with_deg:
            pltpu.sync_copy(ones, rv0)
            plsc.subcore_barrier()

            def dgroup(g, carry):
                pltpu.sync_copy(dst3d.at[sid, pl.ds(g * GC, GC)], dv)

                def dfire(j, c2):
                    pltpu.async_copy(rv0, acc.at[dv.at[cid * (GC // 2) + j]],
                                     ss0, add=True)
                    return c2

                def ddrain(j, c2):
                    pltpu.make_async_copy(
                        rv0, acc.at[dv.at[cid * (GC // 2) + j]], ss0).wait()
                    return c2

                lax.fori_loop(0, GC // 2, dfire, 0, unroll=False)
                lax.fori_loop(0, GC // 2, ddrain, 0, unroll=False)
                return carry

            lax.fori_loop(0, NG, dgroup, 0, unroll=False)
            plsc.subcore_barrier()
            rows_copy(acc, dout.at[cid])
            plsc.subcore_barrier()
            rows_copy(zf, acc)

            @pl.when(sid == 15)
            def _():
                pltpu.sync_copy(zf.at[pl.ds(0, 8)],
                                acc.at[pl.ds(N_NODES, 8)])

        plsc.subcore_barrier()

        rbufs = (rv0, rv1)
        gsems = (sg0, sg1)
        ssems = (ss0, ss1)

        def run(table):
            def fire_g(k, b):
                pltpu.async_copy(table.at[sv.at[k]], rbufs[b], gsems[b])

            def wait_g(k, b):
                pltpu.make_async_copy(table.at[sv.at[k]], rbufs[b],
                                      gsems[b]).wait()

            def fire_s(k, b):
                pltpu.async_copy(rbufs[b], acc.at[dv.at[k]], ssems[b],
                                 add=True)

            def wait_s(k, b):
                pltpu.make_async_copy(rbufs[b], acc.at[dv.at[k]],
                                      ssems[b]).wait()

            def group(g, carry):
                pltpu.sync_copy(src3d.at[sid, pl.ds(g * GC, GC)], sv)
                pltpu.sync_copy(dst3d.at[sid, pl.ds(g * GC, GC)], dv)
                fire_g(0, 0)
                fire_g(1, 1)

                def pair(p, c2):
                    for b in range(2):
                        wait_g(2 * p + b, b)
                        fire_s(2 * p + b, b)
                    for b in range(2):
                        wait_s(2 * p + b, b)
                        fire_g(2 * p + b + 2, b)
                    return c2

                lax.fori_loop(0, GC // 2 - 1, pair, 0, unroll=False)
                for b in range(2):
                    wait_g(GC - 2 + b, b)
                    fire_s(GC - 2 + b, b)
                for b in range(2):
                    wait_s(GC - 2 + b, b)
                return carry

            lax.fori_loop(0, NG, group, 0, unroll=False)

        @pl.when(cid == 0)
        def _():
            run(t0)

        @pl.when(cid == 1)
        def _():
            run(t1)

        plsc.subcore_barrier()

        @pl.when(cid == 0)
        def _():
            rows_copy(acc, out0)

        @pl.when(cid == 1)
        def _():
            rows_copy(acc, out1)

    return body


def _aggregate(t0, t1, src3d, dst3d, zf, ones, with_deg):
    out = _sc_agg_kernel(with_deg)(t0, t1, src3d, dst3d, zf, ones)
    if with_deg:
        a0, a1, dpages = out
        return a0, a1, dpages
    a0, a1 = out
    return a0, a1, None


def kernel(x, edge_index, W0, b0, W1, b1):
    spad = jnp.zeros((N_EPAD - N_EDGES,), jnp.int32)
    dpad = jnp.full((N_EPAD - N_EDGES,), N_NODES, jnp.int32)
    src3d = jnp.concatenate([edge_index[0], spad]).reshape(16, NG * GC, CH)
    dst3d = jnp.concatenate([edge_index[1], dpad]).reshape(16, NG * GC, CH)
    zf = jnp.zeros((N_NODES, 128), jnp.float32)
    ones = jnp.ones((CH, 128), jnp.float32)
    t0, t1 = _tc_pre(x, W0.T, b0.reshape(1, D))
    a0, a1, deg = _aggregate(t0, t1, src3d, dst3d, zf, ones, True)
    t0, t1 = _tc_mid(a0, a1, deg, W1.T, b1.reshape(1, D))
    a0, a1, _ = _aggregate(t0, t1, src3d, dst3d, zf, ones, False)
    return _tc_post(a0, a1, deg)


# speedup vs baseline: 1.0674x; 1.0674x over previous
import functools

import jax
import jax.numpy as jnp
from jax import lax
from jax.experimental import pallas as pl
from jax.experimental.pallas import tpu as pltpu
from jax.experimental.pallas import tpu_sc as plsc

MIN_NORM = 1e-15
EPS = 4e-3
MAX_NORM = 1e6
N_NODES = 10000
N_EDGES = 160000
D = 256

ROWS = 1000


def _col_mask(v):
    col = lax.broadcasted_iota(jnp.int32, v.shape, dimension=v.ndim - 1)
    return col == 0


def _zero_col0(v):
    return jnp.where(_col_mask(v), 0.0, v)


def _sq_rest(v):
    vz = _zero_col0(v)
    return jnp.sum(vz * vz, axis=-1, keepdims=True)


def _dot_rest(a, b):
    p = _zero_col0(a) * _zero_col0(b)
    return jnp.sum(p, axis=-1, keepdims=True)


def _col0(v):
    return v[..., 0:1]


def _set_col0(v, s):
    return jnp.where(_col_mask(v), s, v)


def _cosh_sinh(t):
    e = jnp.exp(t)
    ei = 1.0 / e
    return 0.5 * (e + ei), 0.5 * (e - ei)


def _arccosh(t):
    return jnp.log(t + jnp.sqrt(t * t - 1.0))


def _proj(v):
    x0 = jnp.sqrt(jnp.clip(1.0 + _sq_rest(v), EPS, None))
    return _set_col0(v, x0)


def _expmap0(u):
    n = jnp.clip(jnp.sqrt(_sq_rest(u)), MIN_NORM, None)
    ch, sh = _cosh_sinh(n)
    y = (sh / n) * _zero_col0(u)
    return _proj(_set_col0(y, ch))


def _logmap0(x):
    yn = jnp.clip(jnp.sqrt(_sq_rest(x)), MIN_NORM, None)
    theta = jnp.clip(_col0(x), 1.0 + EPS, None)
    r = (_arccosh(theta) / yn) * _zero_col0(x)
    return _set_col0(r, 0.0)


def _hyp_bias(b_row):
    return _proj(_expmap0(_zero_col0(b_row)))


def _mobius_add(x, yb):
    u = _logmap0(yb)
    x0 = _col0(x)
    y_norm = jnp.clip(jnp.sqrt(_sq_rest(x)), MIN_NORM, None)
    y_unit = _zero_col0(x) / y_norm
    v = _set_col0((1.0 - x0) * y_unit, -y_norm)
    alpha = jnp.sum(y_unit * _zero_col0(u), axis=-1, keepdims=True)
    res = u - alpha * v
    ux = _dot_rest(x, res)
    u0 = ux / jnp.clip(x0, EPS, None)
    res = _set_col0(res, u0)
    mdot = _sq_rest(res) - _col0(res) * _col0(res)
    normu = jnp.sqrt(jnp.clip(mdot, EPS, None))
    normu = jnp.clip(normu, None, MAX_NORM)
    theta = jnp.clip(normu, MIN_NORM, None)
    ch, sh = _cosh_sinh(theta)
    return _proj(ch * x + (sh / theta) * res)


def _hyp_linear_to_tan(x_hyp, Wt, b_row):
    u = _logmap0(x_hyp)
    mu = jnp.dot(u, Wt, preferred_element_type=jnp.float32)
    res = _proj(_expmap0(mu))
    res = _proj(_mobius_add(res, _hyp_bias(b_row)))
    return _logmap0(res)


def _agg_to_hyp(agg):
    h = _proj(_expmap0(agg))
    xt = jax.nn.relu(_logmap0(h))
    xt = _set_col0(xt, 0.0)
    return _proj(_expmap0(xt))


def _tc_pre_body(x_ref, wt_ref, b_ref, o0_ref, o1_ref):
    x = x_ref[...]
    x_hyp = _expmap0(_zero_col0(x))
    xt = _hyp_linear_to_tan(x_hyp, wt_ref[...], b_ref[...])
    o0_ref[...] = xt[:, :128]
    o1_ref[...] = xt[:, 128:]


def _tc_mid_body(a0_ref, a1_ref, deg_ref, wt_ref, b_ref, o0_ref, o1_ref):
    agg = jnp.concatenate([a0_ref[...], a1_ref[...]], axis=-1)
    deg = deg_ref[0, :, 0:1] + deg_ref[1, :, 0:1]
    agg = agg / jnp.clip(deg, 1.0, None)
    h = _agg_to_hyp(agg)
    xt = _hyp_linear_to_tan(h, wt_ref[...], b_ref[...])
    o0_ref[...] = xt[:, :128]
    o1_ref[...] = xt[:, 128:]


def _tc_post_body(a0_ref, a1_ref, deg_ref, o_ref):
    agg = jnp.concatenate([a0_ref[...], a1_ref[...]], axis=-1)
    deg = deg_ref[0, :, 0:1] + deg_ref[1, :, 0:1]
    agg = agg / jnp.clip(deg, 1.0, None)
    h = _agg_to_hyp(agg)
    out = _logmap0(h)
    o_ref[...] = _set_col0(out, 0.0)


def _row_spec(width):
    return pl.BlockSpec((ROWS, width), lambda i: (i, 0))


def _full_spec(shape):
    return pl.BlockSpec(shape, lambda i: tuple(0 for _ in shape))


def _deg_spec():
    return pl.BlockSpec((2, ROWS, 128), lambda i: (0, i, 0))


def _tc_pre(x, Wt, b_row):
    grid = N_NODES // ROWS
    return pl.pallas_call(
        _tc_pre_body,
        grid=(grid,),
        in_specs=[_row_spec(D), _full_spec((D, D)), _full_spec((1, D))],
        out_specs=[_row_spec(128), _row_spec(128)],
        out_shape=[jax.ShapeDtypeStruct((N_NODES, 128), jnp.float32)] * 2,
    )(x, Wt, b_row)


def _tc_mid(a0, a1, deg, Wt, b_row):
    grid = N_NODES // ROWS
    return pl.pallas_call(
        _tc_mid_body,
        grid=(grid,),
        in_specs=[_row_spec(128), _row_spec(128), _deg_spec(),
                  _full_spec((D, D)), _full_spec((1, D))],
        out_specs=[_row_spec(128), _row_spec(128)],
        out_shape=[jax.ShapeDtypeStruct((N_NODES, 128), jnp.float32)] * 2,
    )(a0, a1, deg, Wt, b_row)


def _tc_post(a0, a1, deg):
    grid = N_NODES // ROWS
    return pl.pallas_call(
        _tc_post_body,
        grid=(grid,),
        in_specs=[_row_spec(128), _row_spec(128), _deg_spec()],
        out_specs=_row_spec(D),
        out_shape=jax.ShapeDtypeStruct((N_NODES, D), jnp.float32),
    )(a0, a1, deg)


CH = 128
GC = 16
NG = 5
EPT = NG * GC * CH
N_EPAD = 16 * EPT
ACC_ROWS = N_NODES + 8
R_MAIN = 624
R_TAIL = N_NODES - 16 * R_MAIN


def _sc_agg_kernel(with_deg):
    mesh = plsc.VectorSubcoreMesh(core_axis_name="c", subcore_axis_name="s",
                                  num_cores=2, num_subcores=16)
    out_type = [jax.ShapeDtypeStruct((N_NODES, 128), jnp.float32)] * 2
    scratch = [
        pltpu.VMEM_SHARED((ACC_ROWS, 128), jnp.float32),
        pltpu.VMEM((GC, CH), jnp.int32),
        pltpu.VMEM((GC, CH), jnp.int32),
        pltpu.VMEM((CH, 128), jnp.float32),
        pltpu.VMEM((CH, 128), jnp.float32),
        pltpu.SemaphoreType.DMA,
        pltpu.SemaphoreType.DMA,
        pltpu.SemaphoreType.DMA,
        pltpu.SemaphoreType.DMA,
    ]
    if with_deg:
        out_type.append(jax.ShapeDtypeStruct((2, N_NODES, 128), jnp.float32))

    @functools.partial(pl.kernel, mesh=mesh, out_type=out_type,
                       scratch_types=scratch)
    def body(t0, t1, src3d, dst3d, zf, ones, out0, out1, *rest):
        if with_deg:
            dout, acc, sv, dv, rv0, rv1, sg0, sg1, ss0, ss1 = rest
        else:
            acc, sv, dv, rv0, rv1, sg0, sg1, ss0, ss1 = rest
            dout = None
        cid = lax.axis_index("c")
        sid = lax.axis_index("s")

        def rows_copy(a, b):
            r0 = sid * R_MAIN
            pltpu.sync_copy(a.at[pl.ds(r0, R_MAIN)], b.at[pl.ds(r0, R_MAIN)])

            @pl.when(sid == 15)
            def _():
                t0_ = 16 * R_MAIN
                pltpu.sync_copy(a.at[pl.ds(t0_, R_TAIL)],
                                b.at[pl.ds(t0_, R_TAIL)])

        rows_copy(zf, acc)

        @pl.when(sid == 15)
        def _():
            pltpu.sync_copy(zf.at[pl.ds(0, 8)],
                            acc.at[pl.ds(N_NODES, 8)])

        if with_deg:
            pltpu.sync_copy(ones, rv0)
            plsc.subcore_barrier()

            def dgroup(g, carry):
                pltpu.sync_copy(dst3d.at[sid, pl.ds(g * GC, GC)], dv)

                def dfire(j, c2):
                    pltpu.async_copy(rv0, acc.at[dv.at[cid * (GC // 2) + j]],
                                     ss0, add=True)
                    return c2

                def ddrain(j, c2):
                    pltpu.make_async_copy(
                        rv0, acc.at[dv.at[cid * (GC // 2) + j]], ss0).wait()
                    return c2

                lax.fori_loop(0, GC // 2, dfire, 0, unroll=False)
                lax.fori_loop(0, GC // 2, ddrain, 0, unroll=False)
                return carry

            lax.fori_loop(0, NG, dgroup, 0, unroll=False)
            plsc.subcore_barrier()
            rows_copy(acc, dout.at[cid])
            plsc.subcore_barrier()
            rows_copy(zf, acc)

            @pl.when(sid == 15)
            def _():
                pltpu.sync_copy(zf.at[pl.ds(0, 8)],
                                acc.at[pl.ds(N_NODES, 8)])

        plsc.subcore_barrier()

        rbufs = (rv0, rv1)
        gsems = (sg0, sg1)
        ssems = (ss0, ss1)

        def run(table):
            def fire_g(k, b):
                pltpu.async_copy(table.at[sv.at[k]], rbufs[b], gsems[b])

            def wait_g(k, b):
                pltpu.make_async_copy(table.at[sv.at[k]], rbufs[b],
                                      gsems[b]).wait()

            def fire_s(k, b):
                pltpu.async_copy(rbufs[b], acc.at[dv.at[k]], ssems[b],
                                 add=True)

            def wait_s(k, b):
                pltpu.make_async_copy(rbufs[b], acc.at[dv.at[k]],
                                      ssems[b]).wait()

            def group(g, carry):
                pltpu.sync_copy(src3d.at[sid, pl.ds(g * GC, GC)], sv)
                pltpu.sync_copy(dst3d.at[sid, pl.ds(g * GC, GC)], dv)
                fire_g(0, 0)
                fire_g(1, 1)

                def pair(p, c2):
                    for b in range(2):
                        wait_g(2 * p + b, b)
                        fire_s(2 * p + b, b)
                        wait_s(2 * p + b, b)
                        fire_g(2 * p + b + 2, b)
                    return c2

                lax.fori_loop(0, GC // 2 - 1, pair, 0, unroll=False)
                for b in range(2):
                    wait_g(GC - 2 + b, b)
                    fire_s(GC - 2 + b, b)
                    wait_s(GC - 2 + b, b)
                return carry

            lax.fori_loop(0, NG, group, 0, unroll=False)

        @pl.when(cid == 0)
        def _():
            run(t0)

        @pl.when(cid == 1)
        def _():
            run(t1)

        plsc.subcore_barrier()

        @pl.when(cid == 0)
        def _():
            rows_copy(acc, out0)

        @pl.when(cid == 1)
        def _():
            rows_copy(acc, out1)

    return body


def _aggregate(t0, t1, src3d, dst3d, zf, ones, with_deg):
    out = _sc_agg_kernel(with_deg)(t0, t1, src3d, dst3d, zf, ones)
    if with_deg:
        a0, a1, dpages = out
        return a0, a1, dpages
    a0, a1 = out
    return a0, a1, None


def kernel(x, edge_index, W0, b0, W1, b1):
    spad = jnp.zeros((N_EPAD - N_EDGES,), jnp.int32)
    dpad = jnp.full((N_EPAD - N_EDGES,), N_NODES, jnp.int32)
    src3d = jnp.concatenate([edge_index[0], spad]).reshape(16, NG * GC, CH)
    dst3d = jnp.concatenate([edge_index[1], dpad]).reshape(16, NG * GC, CH)
    zf = jnp.zeros((N_NODES, 128), jnp.float32)
    ones = jnp.ones((CH, 128), jnp.float32)
    t0, t1 = _tc_pre(x, W0.T, b0.reshape(1, D))
    a0, a1, deg = _aggregate(t0, t1, src3d, dst3d, zf, ones, True)
    t0, t1 = _tc_mid(a0, a1, deg, W1.T, b1.reshape(1, D))
    a0, a1, _ = _aggregate(t0, t1, src3d, dst3d, zf, ones, False)
    return _tc_post(a0, a1, deg)


# speedup vs baseline: 1.0682x; 1.0007x over previous
import functools

import jax
import jax.numpy as jnp
from jax import lax
from jax.experimental import pallas as pl
from jax.experimental.pallas import tpu as pltpu
from jax.experimental.pallas import tpu_sc as plsc

MIN_NORM = 1e-15
EPS = 4e-3
MAX_NORM = 1e6
N_NODES = 10000
N_EDGES = 160000
D = 256

ROWS = 1000


def _col_mask(v):
    col = lax.broadcasted_iota(jnp.int32, v.shape, dimension=v.ndim - 1)
    return col == 0


def _zero_col0(v):
    return jnp.where(_col_mask(v), 0.0, v)


def _sq_rest(v):
    vz = _zero_col0(v)
    return jnp.sum(vz * vz, axis=-1, keepdims=True)


def _dot_rest(a, b):
    p = _zero_col0(a) * _zero_col0(b)
    return jnp.sum(p, axis=-1, keepdims=True)


def _col0(v):
    return v[..., 0:1]


def _set_col0(v, s):
    return jnp.where(_col_mask(v), s, v)


def _cosh_sinh(t):
    e = jnp.exp(t)
    ei = 1.0 / e
    return 0.5 * (e + ei), 0.5 * (e - ei)


def _arccosh(t):
    return jnp.log(t + jnp.sqrt(t * t - 1.0))


def _proj(v):
    x0 = jnp.sqrt(jnp.clip(1.0 + _sq_rest(v), EPS, None))
    return _set_col0(v, x0)


def _expmap0(u):
    n = jnp.clip(jnp.sqrt(_sq_rest(u)), MIN_NORM, None)
    ch, sh = _cosh_sinh(n)
    y = (sh / n) * _zero_col0(u)
    return _proj(_set_col0(y, ch))


def _logmap0(x):
    yn = jnp.clip(jnp.sqrt(_sq_rest(x)), MIN_NORM, None)
    theta = jnp.clip(_col0(x), 1.0 + EPS, None)
    r = (_arccosh(theta) / yn) * _zero_col0(x)
    return _set_col0(r, 0.0)


def _hyp_bias(b_row):
    return _proj(_expmap0(_zero_col0(b_row)))


def _mobius_add(x, yb):
    u = _logmap0(yb)
    x0 = _col0(x)
    y_norm = jnp.clip(jnp.sqrt(_sq_rest(x)), MIN_NORM, None)
    y_unit = _zero_col0(x) / y_norm
    v = _set_col0((1.0 - x0) * y_unit, -y_norm)
    alpha = jnp.sum(y_unit * _zero_col0(u), axis=-1, keepdims=True)
    res = u - alpha * v
    ux = _dot_rest(x, res)
    u0 = ux / jnp.clip(x0, EPS, None)
    res = _set_col0(res, u0)
    mdot = _sq_rest(res) - _col0(res) * _col0(res)
    normu = jnp.sqrt(jnp.clip(mdot, EPS, None))
    normu = jnp.clip(normu, None, MAX_NORM)
    theta = jnp.clip(normu, MIN_NORM, None)
    ch, sh = _cosh_sinh(theta)
    return _proj(ch * x + (sh / theta) * res)


def _hyp_linear_to_tan(x_hyp, Wt, b_row):
    u = _logmap0(x_hyp)
    mu = jnp.dot(u, Wt, preferred_element_type=jnp.float32)
    res = _proj(_expmap0(mu))
    res = _proj(_mobius_add(res, _hyp_bias(b_row)))
    return _logmap0(res)


def _agg_to_hyp(agg):
    h = _proj(_expmap0(agg))
    xt = jax.nn.relu(_logmap0(h))
    xt = _set_col0(xt, 0.0)
    return _proj(_expmap0(xt))


def _tc_pre_body(x_ref, wt_ref, b_ref, o0_ref, o1_ref):
    x = x_ref[...]
    x_hyp = _expmap0(_zero_col0(x))
    xt = _hyp_linear_to_tan(x_hyp, wt_ref[...], b_ref[...])
    o0_ref[...] = xt[:, :128]
    o1_ref[...] = xt[:, 128:]


def _tc_mid_body(a0_ref, a1_ref, deg_ref, wt_ref, b_ref, o0_ref, o1_ref):
    agg = jnp.concatenate([a0_ref[...], a1_ref[...]], axis=-1)
    deg = deg_ref[0, :, 0:1] + deg_ref[1, :, 0:1]
    agg = agg / jnp.clip(deg, 1.0, None)
    h = _agg_to_hyp(agg)
    xt = _hyp_linear_to_tan(h, wt_ref[...], b_ref[...])
    o0_ref[...] = xt[:, :128]
    o1_ref[...] = xt[:, 128:]


def _tc_post_body(a0_ref, a1_ref, deg_ref, o_ref):
    agg = jnp.concatenate([a0_ref[...], a1_ref[...]], axis=-1)
    deg = deg_ref[0, :, 0:1] + deg_ref[1, :, 0:1]
    agg = agg / jnp.clip(deg, 1.0, None)
    h = _agg_to_hyp(agg)
    out = _logmap0(h)
    o_ref[...] = _set_col0(out, 0.0)


def _row_spec(width):
    return pl.BlockSpec((ROWS, width), lambda i: (i, 0))


def _full_spec(shape):
    return pl.BlockSpec(shape, lambda i: tuple(0 for _ in shape))


def _deg_spec():
    return pl.BlockSpec((2, ROWS, 128), lambda i: (0, i, 0))


def _tc_pre(x, Wt, b_row):
    grid = N_NODES // ROWS
    return pl.pallas_call(
        _tc_pre_body,
        grid=(grid,),
        in_specs=[_row_spec(D), _full_spec((D, D)), _full_spec((1, D))],
        out_specs=[_row_spec(128), _row_spec(128)],
        out_shape=[jax.ShapeDtypeStruct((N_NODES, 128), jnp.float32)] * 2,
    )(x, Wt, b_row)


def _tc_mid(a0, a1, deg, Wt, b_row):
    grid = N_NODES // ROWS
    return pl.pallas_call(
        _tc_mid_body,
        grid=(grid,),
        in_specs=[_row_spec(128), _row_spec(128), _deg_spec(),
                  _full_spec((D, D)), _full_spec((1, D))],
        out_specs=[_row_spec(128), _row_spec(128)],
        out_shape=[jax.ShapeDtypeStruct((N_NODES, 128), jnp.float32)] * 2,
    )(a0, a1, deg, Wt, b_row)


def _tc_post(a0, a1, deg):
    grid = N_NODES // ROWS
    return pl.pallas_call(
        _tc_post_body,
        grid=(grid,),
        in_specs=[_row_spec(128), _row_spec(128), _deg_spec()],
        out_specs=_row_spec(D),
        out_shape=jax.ShapeDtypeStruct((N_NODES, D), jnp.float32),
    )(a0, a1, deg)


CH = 128
GC = 16
NG = 5
EPT = NG * GC * CH
N_EPAD = 16 * EPT
ACC_ROWS = N_NODES + 8
R_MAIN = 624
R_TAIL = N_NODES - 16 * R_MAIN


def _sc_agg_kernel(with_deg):
    mesh = plsc.VectorSubcoreMesh(core_axis_name="c", subcore_axis_name="s",
                                  num_cores=2, num_subcores=16)
    out_type = [jax.ShapeDtypeStruct((N_NODES, 128), jnp.float32)] * 2
    scratch = [
        pltpu.VMEM_SHARED((ACC_ROWS, 128), jnp.float32),
        pltpu.VMEM((GC, CH), jnp.int32),
        pltpu.VMEM((GC, CH), jnp.int32),
        pltpu.VMEM((CH, 128), jnp.float32),
        pltpu.VMEM((CH, 128), jnp.float32),
        pltpu.SemaphoreType.DMA,
        pltpu.SemaphoreType.DMA,
        pltpu.SemaphoreType.DMA,
        pltpu.SemaphoreType.DMA,
    ]
    if with_deg:
        out_type.append(jax.ShapeDtypeStruct((2, N_NODES, 128), jnp.float32))

    @functools.partial(pl.kernel, mesh=mesh, out_type=out_type,
                       scratch_types=scratch)
    def body(t0, t1, src3d, dst3d, zf, ones, out0, out1, *rest):
        if with_deg:
            dout, acc, sv, dv, rv0, rv1, sg0, sg1, ss0, ss1 = rest
        else:
            acc, sv, dv, rv0, rv1, sg0, sg1, ss0, ss1 = rest
            dout = None
        cid = lax.axis_index("c")
        sid = lax.axis_index("s")

        def rows_copy(a, b):
            r0 = sid * R_MAIN
            pltpu.sync_copy(a.at[pl.ds(r0, R_MAIN)], b.at[pl.ds(r0, R_MAIN)])

            @pl.when(sid == 15)
            def _():
                t0_ = 16 * R_MAIN
                pltpu.sync_copy(a.at[pl.ds(t0_, R_TAIL)],
                                b.at[pl.ds(t0_, R_TAIL)])

        rows_copy(zf, acc)

        @pl.when(sid == 15)
        def _():
            pltpu.sync_copy(zf.at[pl.ds(0, 8)],
                            acc.at[pl.ds(N_NODES, 8)])

        if with_deg:
            pltpu.sync_copy(ones, rv0)
            plsc.subcore_barrier()

            def dgroup(g, carry):
                pltpu.sync_copy(dst3d.at[sid, pl.ds(g * GC, GC)], dv)

                def dfire(j, c2):
                    pltpu.async_copy(rv0, acc.at[dv.at[cid * (GC // 2) + j]],
                                     ss0, add=True)
                    return c2

                lax.fori_loop(0, GC // 2, dfire, 0, unroll=False)
                pltpu.make_async_copy(zf.at[pl.ds(0, (GC // 2) * CH)],
                                      acc.at[pl.ds(0, (GC // 2) * CH)],
                                      ss0).wait()
                return carry

            lax.fori_loop(0, NG, dgroup, 0, unroll=False)
            plsc.subcore_barrier()
            rows_copy(acc, dout.at[cid])
            plsc.subcore_barrier()
            rows_copy(zf, acc)

            @pl.when(sid == 15)
            def _():
                pltpu.sync_copy(zf.at[pl.ds(0, 8)],
                                acc.at[pl.ds(N_NODES, 8)])

        plsc.subcore_barrier()

        rbufs = (rv0, rv1)
        gsems = (sg0, sg1)
        ssems = (ss0, ss1)

        def run(table):
            def fire_g(k, b):
                pltpu.async_copy(table.at[sv.at[k]], rbufs[b], gsems[b])

            def wait_g(k, b):
                pltpu.make_async_copy(zf.at[pl.ds(0, CH)], rbufs[b],
                                      gsems[b]).wait()

            def fire_s(k, b):
                pltpu.async_copy(rbufs[b], acc.at[dv.at[k]], ssems[b],
                                 add=True)

            def wait_s(k, b):
                pltpu.make_async_copy(zf.at[pl.ds(0, CH)], rbufs[b],
                                      ssems[b]).wait()

            def group(g, carry):
                pltpu.sync_copy(src3d.at[sid, pl.ds(g * GC, GC)], sv)
                pltpu.sync_copy(dst3d.at[sid, pl.ds(g * GC, GC)], dv)
                fire_g(0, 0)
                fire_g(1, 1)

                def pair(p, c2):
                    for b in range(2):
                        wait_g(2 * p + b, b)
                        fire_s(2 * p + b, b)
                        wait_s(2 * p + b, b)
                        fire_g(2 * p + b + 2, b)
                    return c2

                lax.fori_loop(0, GC // 2 - 1, pair, 0, unroll=2)
                for b in range(2):
                    wait_g(GC - 2 + b, b)
                    fire_s(GC - 2 + b, b)
                    wait_s(GC - 2 + b, b)
                return carry

            lax.fori_loop(0, NG, group, 0, unroll=False)

        @pl.when(cid == 0)
        def _():
            run(t0)

        @pl.when(cid == 1)
        def _():
            run(t1)

        plsc.subcore_barrier()

        @pl.when(cid == 0)
        def _():
            rows_copy(acc, out0)

        @pl.when(cid == 1)
        def _():
            rows_copy(acc, out1)

    return body


def _aggregate(t0, t1, src3d, dst3d, zf, ones, with_deg):
    out = _sc_agg_kernel(with_deg)(t0, t1, src3d, dst3d, zf, ones)
    if with_deg:
        a0, a1, dpages = out
        return a0, a1, dpages
    a0, a1 = out
    return a0, a1, None


def kernel(x, edge_index, W0, b0, W1, b1):
    spad = jnp.zeros((N_EPAD - N_EDGES,), jnp.int32)
    dpad = jnp.full((N_EPAD - N_EDGES,), N_NODES, jnp.int32)
    src3d = jnp.concatenate([edge_index[0], spad]).reshape(16, NG * GC, CH)
    dst3d = jnp.concatenate([edge_index[1], dpad]).reshape(16, NG * GC, CH)
    zf = jnp.zeros((N_NODES, 128), jnp.float32)
    ones = jnp.ones((CH, 128), jnp.float32)
    t0, t1 = _tc_pre(x, W0.T, b0.reshape(1, D))
    a0, a1, deg = _aggregate(t0, t1, src3d, dst3d, zf, ones, True)
    t0, t1 = _tc_mid(a0, a1, deg, W1.T, b1.reshape(1, D))
    a0, a1, _ = _aggregate(t0, t1, src3d, dst3d, zf, ones, False)
    return _tc_post(a0, a1, deg)
